# baseline (speedup 1.0000x reference)
import dataclasses
import functools

import jax
import jax.numpy as jnp
from jax import lax
from jax.experimental import pallas as pl
from jax.experimental.pallas import tpu as pltpu
from jax.experimental.pallas import tpu_sc as plsc

B, N, S = 8, 4096, 1024
C1, C2 = 256, 512
TN = 512
NT = N // TN
NSTEPS = B * NT
NPTS = float(B * N)

NC, L = 2, 16
NW = 32
PW = (B * N) // NW
CH = 32
NCH = PW // CH

_sc_mesh = plsc.VectorSubcoreMesh(core_axis_name="c", subcore_axis_name="s")
_cp = pltpu.CompilerParams()
if "needs_layout_passes" in pltpu.CompilerParams.__dataclass_fields__:
    _cp = dataclasses.replace(_cp, needs_layout_passes=False)


def _dist_body(xyz1_ref, xyz2t_ref, gid_ref, wt_ref):
    b = pl.program_id(0)
    x1 = xyz1_ref[0]
    z2 = xyz2t_ref[0]
    s_src = (x1[0:1, :] * x1[0:1, :] + x1[1:2, :] * x1[1:2, :]
             + x1[2:3, :] * x1[2:3, :])
    d_dst = (z2[:, 0:1] * z2[:, 0:1] + z2[:, 1:2] * z2[:, 1:2]
             + z2[:, 2:3] * z2[:, 2:3])
    cross = jnp.dot(z2, x1, preferred_element_type=jnp.float32)
    dist = d_dst + s_src - 2.0 * cross

    iota = lax.broadcasted_iota(jnp.int32, (S, TN), 0)
    recips = []
    idxs = []
    for _ in range(3):
        mval = jnp.min(dist, axis=0, keepdims=True)
        eq = dist == mval
        idx = jnp.min(jnp.where(eq, iota, jnp.int32(S)), axis=0,
                      keepdims=True)
        sel = iota == idx
        recips.append(1.0 / (mval + 1e-8))
        idxs.append(idx)
        dist = jnp.where(sel, jnp.float32(jnp.inf), dist)
    norm = recips[0] + recips[1] + recips[2]
    for k in range(3):
        gid_ref[0, k:k+1, :] = idxs[k] + b * S
        wt_ref[0, k:k+1, :] = recips[k] / norm


@functools.partial(
    pl.kernel, mesh=_sc_mesh, compiler_params=_cp,
    out_type=jax.ShapeDtypeStruct((B, N, C2), jnp.float32),
    scratch_types=[
        pltpu.VMEM((CH,), jnp.int32), pltpu.VMEM((CH,), jnp.int32),
        pltpu.VMEM((CH,), jnp.int32),
        pltpu.VMEM((CH,), jnp.float32), pltpu.VMEM((CH,), jnp.float32),
        pltpu.VMEM((CH,), jnp.float32),
        pltpu.VMEM((CH, C2), jnp.float32), pltpu.VMEM((CH, C2), jnp.float32),
        pltpu.VMEM((CH, C2), jnp.float32), pltpu.VMEM((CH, C2), jnp.float32),
        pltpu.SemaphoreType.DMA, pltpu.SemaphoreType.DMA,
        pltpu.SemaphoreType.DMA,
    ])
def _sc_gather(table_hbm, gid_hbm, wt_hbm, out_hbm,
               i0, i1, i2, w0, w1, w2, r0, r1, r2, ov, s0, s1, s2):
    wid = lax.axis_index("s") * NC + lax.axis_index("c")
    b = wid // 4
    nb0 = (wid % 4) * PW

    @pl.loop(0, NCH)
    def _(g):
        nb = nb0 + g * CH
        pltpu.sync_copy(gid_hbm.at[b, 0, pl.ds(nb, CH)], i0)
        pltpu.sync_copy(gid_hbm.at[b, 1, pl.ds(nb, CH)], i1)
        pltpu.sync_copy(gid_hbm.at[b, 2, pl.ds(nb, CH)], i2)
        pltpu.sync_copy(wt_hbm.at[b, 0, pl.ds(nb, CH)], w0)
        pltpu.sync_copy(wt_hbm.at[b, 1, pl.ds(nb, CH)], w1)
        pltpu.sync_copy(wt_hbm.at[b, 2, pl.ds(nb, CH)], w2)
        c0 = pltpu.async_copy(table_hbm.at[i0], r0, s0)
        c1 = pltpu.async_copy(table_hbm.at[i1], r1, s1)
        c2 = pltpu.async_copy(table_hbm.at[i2], r2, s2)
        c0.wait()
        c1.wait()
        c2.wait()

        @pl.loop(0, CH)
        def _(i):
            wb0 = plsc.load_gather(w0, [jnp.full((L,), i, jnp.int32)])
            wb1 = plsc.load_gather(w1, [jnp.full((L,), i, jnp.int32)])
            wb2 = plsc.load_gather(w2, [jnp.full((L,), i, jnp.int32)])

            @pl.loop(0, C2, step=L)
            def _(j):
                ov[i, pl.ds(j, L)] = (wb0 * r0[i, pl.ds(j, L)]
                                      + wb1 * r1[i, pl.ds(j, L)]
                                      + wb2 * r2[i, pl.ds(j, L)])

        pltpu.sync_copy(ov, out_hbm.at[b, pl.ds(nb, CH)])


def _acc_stats(step, y, acc1, acc2, sto_ref):
    @pl.when(step == 0)
    def _():
        acc1[...] = y
        acc2[...] = y * y

    @pl.when(step != 0)
    def _():
        acc1[...] += y
        acc2[...] += y * y

    @pl.when(step == NSTEPS - 1)
    def _():
        sto_ref[:, 0:1] = jnp.sum(acc1[...], axis=1, keepdims=True)
        sto_ref[:, 1:2] = jnp.sum(acc2[...], axis=1, keepdims=True)


def _l0_body(p1_ref, it_ref, w0a_ref, w0b_ref, b0_ref,
             y_ref, st_ref, acc1, acc2):
    step = pl.program_id(0) * NT + pl.program_id(1)
    y = (jnp.dot(w0a_ref[...], p1_ref[0], preferred_element_type=jnp.float32)
         + lax.dot_general(w0b_ref[...], it_ref[0],
                           (((1,), (1,)), ((), ())),
                           preferred_element_type=jnp.float32)
         + b0_ref[...])
    y_ref[0] = y
    _acc_stats(step, y, acc1, acc2, st_ref)


def _bn_relu(st_ref, g_ref, beta_ref):
    mean = st_ref[:, 0:1] * (1.0 / NPTS)
    var = st_ref[:, 1:2] * (1.0 / NPTS) - mean * mean
    a = g_ref[...] * lax.rsqrt(var + 1e-5)
    c = beta_ref[...] - a * mean
    return a, c


def _layer_body(y_ref, st_ref, g_ref, beta_ref, w_ref, b_ref,
                yo_ref, sto_ref, acc1, acc2):
    step = pl.program_id(0) * NT + pl.program_id(1)
    a, c = _bn_relu(st_ref, g_ref, beta_ref)
    x = jnp.maximum(a * y_ref[0] + c, 0.0)
    y = jnp.dot(w_ref[...], x, preferred_element_type=jnp.float32) + b_ref[...]
    yo_ref[0] = y
    _acc_stats(step, y, acc1, acc2, sto_ref)


def _final_body(y_ref, st_ref, g_ref, beta_ref, o_ref):
    a, c = _bn_relu(st_ref, g_ref, beta_ref)
    o_ref[0] = jnp.maximum(a * y_ref[0] + c, 0.0)


def _col(v):
    return v.reshape(-1, 1)


def kernel(xyz1, xyz2, points1, points2, W0, b0, g0, beta0,
           W1, b1, g1, beta1, W2, b2, g2, beta2):
    xyz2t = jnp.transpose(xyz2, (0, 2, 1))
    p2flat = jnp.transpose(points2, (0, 2, 1)).reshape(B * S, C2)
    w0a, w0b = W0[:, :C1], W0[:, C1:]

    full = lambda shp: pl.BlockSpec(shp, lambda b, n: (0, 0))
    tile = lambda c: pl.BlockSpec((1, c, TN), lambda b, n: (b, 0, n))
    perb = lambda r, c: pl.BlockSpec((1, r, c), lambda b, n: (b, 0, 0))

    gid, wt = pl.pallas_call(
        _dist_body,
        grid=(B, NT),
        in_specs=[tile(3), perb(S, 3)],
        out_specs=[tile(3), tile(3)],
        out_shape=[jax.ShapeDtypeStruct((B, 3, N), jnp.int32),
                   jax.ShapeDtypeStruct((B, 3, N), jnp.float32)],
    )(xyz1, xyz2t)

    interp = _sc_gather(p2flat, gid, wt)

    y0, st0 = pl.pallas_call(
        _l0_body,
        grid=(B, NT),
        in_specs=[tile(C1), pl.BlockSpec((1, TN, C2), lambda b, n: (b, n, 0)),
                  full((512, C1)), full((512, C2)), full((512, 1))],
        out_specs=[tile(512), full((512, 2))],
        out_shape=[jax.ShapeDtypeStruct((B, 512, N), jnp.float32),
                   jax.ShapeDtypeStruct((512, 2), jnp.float32)],
        scratch_shapes=[pltpu.VMEM((512, TN), jnp.float32),
                        pltpu.VMEM((512, TN), jnp.float32)],
    )(points1, interp, w0a, w0b, _col(b0))

    def layer(y, st, g, beta, W, bias, c_in, c_out):
        return pl.pallas_call(
            _layer_body,
            grid=(B, NT),
            in_specs=[tile(c_in), full((c_in, 2)), full((c_in, 1)),
                      full((c_in, 1)), full((c_out, c_in)), full((c_out, 1))],
            out_specs=[tile(c_out), full((c_out, 2))],
            out_shape=[jax.ShapeDtypeStruct((B, c_out, N), jnp.float32),
                       jax.ShapeDtypeStruct((c_out, 2), jnp.float32)],
            scratch_shapes=[pltpu.VMEM((c_out, TN), jnp.float32),
                            pltpu.VMEM((c_out, TN), jnp.float32)],
        )(y, st, _col(g), _col(beta), W, _col(bias))

    y1, st1 = layer(y0, st0, g0, beta0, W1, b1, 512, 512)
    y2, st2 = layer(y1, st1, g1, beta1, W2, b2, 512, 256)

    out = pl.pallas_call(
        _final_body,
        grid=(B, NT),
        in_specs=[tile(256), full((256, 2)), full((256, 1)), full((256, 1))],
        out_specs=tile(256),
        out_shape=jax.ShapeDtypeStruct((B, 256, N), jnp.float32),
    )(y2, st2, _col(g2), _col(beta2))
    return out


# speedup vs baseline: 1.1078x; 1.1078x over previous
import dataclasses
import functools

import jax
import jax.numpy as jnp
from jax import lax
from jax.experimental import pallas as pl
from jax.experimental.pallas import tpu as pltpu
from jax.experimental.pallas import tpu_sc as plsc

B, N, S = 8, 4096, 1024
C1, C2 = 256, 512
TN = 512
NT = N // TN
NSTEPS = B * NT
NPTS = float(B * N)

NC, L = 2, 16
NW = 32
PW = (B * N) // NW
CH = 32
NCH = PW // CH

_sc_mesh = plsc.VectorSubcoreMesh(core_axis_name="c", subcore_axis_name="s")
_cp = pltpu.CompilerParams()
if "needs_layout_passes" in pltpu.CompilerParams.__dataclass_fields__:
    _cp = dataclasses.replace(_cp, needs_layout_passes=False)


def _dist_body(xyz1_ref, xyz2t_ref, gid_ref, wt_ref):
    b = pl.program_id(0)
    x1 = xyz1_ref[0]
    z2 = xyz2t_ref[0]
    s_src = (x1[0:1, :] * x1[0:1, :] + x1[1:2, :] * x1[1:2, :]
             + x1[2:3, :] * x1[2:3, :])
    d_dst = (z2[:, 0:1] * z2[:, 0:1] + z2[:, 1:2] * z2[:, 1:2]
             + z2[:, 2:3] * z2[:, 2:3])
    cross = jnp.dot(z2, x1, preferred_element_type=jnp.float32)
    dist = d_dst + s_src - 2.0 * cross

    iota = lax.broadcasted_iota(jnp.int32, (S, TN), 0)
    recips = []
    idxs = []
    for _ in range(3):
        mval = jnp.min(dist, axis=0, keepdims=True)
        eq = dist == mval
        idx = jnp.min(jnp.where(eq, iota, jnp.int32(S)), axis=0,
                      keepdims=True)
        sel = iota == idx
        recips.append(1.0 / (mval + 1e-8))
        idxs.append(idx)
        dist = jnp.where(sel, jnp.float32(jnp.inf), dist)
    norm = recips[0] + recips[1] + recips[2]
    for k in range(3):
        gid_ref[0, k:k+1, :] = idxs[k] + b * S
        wt_ref[0, k:k+1, :] = recips[k] / norm


@functools.partial(
    pl.kernel, mesh=_sc_mesh, compiler_params=_cp,
    out_type=jax.ShapeDtypeStruct((B, N, C2), jnp.float32),
    scratch_types=[
        pltpu.VMEM((3, PW), jnp.int32),
        pltpu.VMEM((3, PW), jnp.float32),
        pltpu.VMEM((CH, C2), jnp.float32), pltpu.VMEM((CH, C2), jnp.float32),
        pltpu.VMEM((CH, C2), jnp.float32), pltpu.VMEM((CH, C2), jnp.float32),
        pltpu.SemaphoreType.DMA, pltpu.SemaphoreType.DMA,
        pltpu.SemaphoreType.DMA,
    ])
def _sc_gather(table_hbm, gid_hbm, wt_hbm, out_hbm,
               ia, wa, r0, r1, r2, ov, s0, s1, s2):
    wid = lax.axis_index("s") * NC + lax.axis_index("c")
    b = wid // 4
    nb0 = (wid % 4) * PW

    pltpu.sync_copy(gid_hbm.at[b, :, pl.ds(nb0, PW)], ia)
    pltpu.sync_copy(wt_hbm.at[b, :, pl.ds(nb0, PW)], wa)

    @pl.loop(0, NCH)
    def _(g):
        nb = nb0 + g * CH
        c0 = pltpu.async_copy(table_hbm.at[ia.at[0, pl.ds(g * CH, CH)]],
                              r0, s0)
        c1 = pltpu.async_copy(table_hbm.at[ia.at[1, pl.ds(g * CH, CH)]],
                              r1, s1)
        c2 = pltpu.async_copy(table_hbm.at[ia.at[2, pl.ds(g * CH, CH)]],
                              r2, s2)
        c0.wait()
        c1.wait()
        c2.wait()

        @pl.loop(0, CH)
        def _(i):
            p = g * CH + i
            wb0 = plsc.load_gather(
                wa, [jnp.full((L,), 0, jnp.int32), jnp.full((L,), p, jnp.int32)])
            wb1 = plsc.load_gather(
                wa, [jnp.full((L,), 1, jnp.int32), jnp.full((L,), p, jnp.int32)])
            wb2 = plsc.load_gather(
                wa, [jnp.full((L,), 2, jnp.int32), jnp.full((L,), p, jnp.int32)])

            @pl.loop(0, C2, step=L)
            def _(j):
                ov[i, pl.ds(j, L)] = (wb0 * r0[i, pl.ds(j, L)]
                                      + wb1 * r1[i, pl.ds(j, L)]
                                      + wb2 * r2[i, pl.ds(j, L)])

        pltpu.sync_copy(ov, out_hbm.at[b, pl.ds(nb, CH)])


def _acc_stats(step, y, acc1, acc2, sto_ref):
    @pl.when(step == 0)
    def _():
        acc1[...] = y
        acc2[...] = y * y

    @pl.when(step != 0)
    def _():
        acc1[...] += y
        acc2[...] += y * y

    @pl.when(step == NSTEPS - 1)
    def _():
        sto_ref[:, 0:1] = jnp.sum(acc1[...], axis=1, keepdims=True)
        sto_ref[:, 1:2] = jnp.sum(acc2[...], axis=1, keepdims=True)


def _l0_body(p1_ref, it_ref, w0a_ref, w0b_ref, b0_ref,
             y_ref, st_ref, acc1, acc2):
    step = pl.program_id(0) * NT + pl.program_id(1)
    y = (jnp.dot(w0a_ref[...], p1_ref[0], preferred_element_type=jnp.float32)
         + lax.dot_general(w0b_ref[...], it_ref[0],
                           (((1,), (1,)), ((), ())),
                           preferred_element_type=jnp.float32)
         + b0_ref[...])
    y_ref[0] = y
    _acc_stats(step, y, acc1, acc2, st_ref)


def _bn_relu(st_ref, g_ref, beta_ref):
    mean = st_ref[:, 0:1] * (1.0 / NPTS)
    var = st_ref[:, 1:2] * (1.0 / NPTS) - mean * mean
    a = g_ref[...] * lax.rsqrt(var + 1e-5)
    c = beta_ref[...] - a * mean
    return a, c


def _layer_body(y_ref, st_ref, g_ref, beta_ref, w_ref, b_ref,
                yo_ref, sto_ref, acc1, acc2):
    step = pl.program_id(0) * NT + pl.program_id(1)
    a, c = _bn_relu(st_ref, g_ref, beta_ref)
    x = jnp.maximum(a * y_ref[0] + c, 0.0)
    y = jnp.dot(w_ref[...], x, preferred_element_type=jnp.float32) + b_ref[...]
    yo_ref[0] = y
    _acc_stats(step, y, acc1, acc2, sto_ref)


def _final_body(y_ref, st_ref, g_ref, beta_ref, o_ref):
    a, c = _bn_relu(st_ref, g_ref, beta_ref)
    o_ref[0] = jnp.maximum(a * y_ref[0] + c, 0.0)


def _col(v):
    return v.reshape(-1, 1)


def kernel(xyz1, xyz2, points1, points2, W0, b0, g0, beta0,
           W1, b1, g1, beta1, W2, b2, g2, beta2):
    xyz2t = jnp.transpose(xyz2, (0, 2, 1))
    p2flat = jnp.transpose(points2, (0, 2, 1)).reshape(B * S, C2)
    w0a, w0b = W0[:, :C1], W0[:, C1:]

    full = lambda shp: pl.BlockSpec(shp, lambda b, n: (0, 0))
    tile = lambda c: pl.BlockSpec((1, c, TN), lambda b, n: (b, 0, n))
    perb = lambda r, c: pl.BlockSpec((1, r, c), lambda b, n: (b, 0, 0))

    gid, wt = pl.pallas_call(
        _dist_body,
        grid=(B, NT),
        in_specs=[tile(3), perb(S, 3)],
        out_specs=[tile(3), tile(3)],
        out_shape=[jax.ShapeDtypeStruct((B, 3, N), jnp.int32),
                   jax.ShapeDtypeStruct((B, 3, N), jnp.float32)],
    )(xyz1, xyz2t)

    interp = _sc_gather(p2flat, gid, wt)

    y0, st0 = pl.pallas_call(
        _l0_body,
        grid=(B, NT),
        in_specs=[tile(C1), pl.BlockSpec((1, TN, C2), lambda b, n: (b, n, 0)),
                  full((512, C1)), full((512, C2)), full((512, 1))],
        out_specs=[tile(512), full((512, 2))],
        out_shape=[jax.ShapeDtypeStruct((B, 512, N), jnp.float32),
                   jax.ShapeDtypeStruct((512, 2), jnp.float32)],
        scratch_shapes=[pltpu.VMEM((512, TN), jnp.float32),
                        pltpu.VMEM((512, TN), jnp.float32)],
    )(points1, interp, w0a, w0b, _col(b0))

    def layer(y, st, g, beta, W, bias, c_in, c_out):
        return pl.pallas_call(
            _layer_body,
            grid=(B, NT),
            in_specs=[tile(c_in), full((c_in, 2)), full((c_in, 1)),
                      full((c_in, 1)), full((c_out, c_in)), full((c_out, 1))],
            out_specs=[tile(c_out), full((c_out, 2))],
            out_shape=[jax.ShapeDtypeStruct((B, c_out, N), jnp.float32),
                       jax.ShapeDtypeStruct((c_out, 2), jnp.float32)],
            scratch_shapes=[pltpu.VMEM((c_out, TN), jnp.float32),
                            pltpu.VMEM((c_out, TN), jnp.float32)],
        )(y, st, _col(g), _col(beta), W, _col(bias))

    y1, st1 = layer(y0, st0, g0, beta0, W1, b1, 512, 512)
    y2, st2 = layer(y1, st1, g1, beta1, W2, b2, 512, 256)

    out = pl.pallas_call(
        _final_body,
        grid=(B, NT),
        in_specs=[tile(256), full((256, 2)), full((256, 1)), full((256, 1))],
        out_specs=tile(256),
        out_shape=jax.ShapeDtypeStruct((B, 256, N), jnp.float32),
    )(y2, st2, _col(g2), _col(beta2))
    return out


# speedup vs baseline: 1.2593x; 1.1367x over previous
import dataclasses
import functools

import jax
import jax.numpy as jnp
from jax import lax
from jax.experimental import pallas as pl
from jax.experimental.pallas import tpu as pltpu
from jax.experimental.pallas import tpu_sc as plsc

B, N, S = 8, 4096, 1024
C1, C2 = 256, 512
TN = 512
NT = N // TN
NSTEPS = B * NT
NPTS = float(B * N)

NC, L = 2, 16
NW = 32
PW = (B * N) // NW
CH = 32
NCH = PW // CH

_sc_mesh = plsc.VectorSubcoreMesh(core_axis_name="c", subcore_axis_name="s")
_cp = pltpu.CompilerParams()
if "needs_layout_passes" in pltpu.CompilerParams.__dataclass_fields__:
    _cp = dataclasses.replace(_cp, needs_layout_passes=False)


def _dist_body(xyz1_ref, xyz2t_ref, gid_ref, wt_ref):
    b = pl.program_id(0)
    x1 = xyz1_ref[0]
    z2 = xyz2t_ref[0]
    s_src = (x1[0:1, :] * x1[0:1, :] + x1[1:2, :] * x1[1:2, :]
             + x1[2:3, :] * x1[2:3, :])
    d_dst = (z2[:, 0:1] * z2[:, 0:1] + z2[:, 1:2] * z2[:, 1:2]
             + z2[:, 2:3] * z2[:, 2:3])
    cross = jnp.dot(z2, x1, preferred_element_type=jnp.float32)
    dist = d_dst + s_src - 2.0 * cross

    iota = lax.broadcasted_iota(jnp.int32, (S, TN), 0)
    recips = []
    idxs = []
    for _ in range(3):
        mval = jnp.min(dist, axis=0, keepdims=True)
        eq = dist == mval
        idx = jnp.min(jnp.where(eq, iota, jnp.int32(S)), axis=0,
                      keepdims=True)
        sel = iota == idx
        recips.append(1.0 / (mval + 1e-8))
        idxs.append(idx)
        dist = jnp.where(sel, jnp.float32(jnp.inf), dist)
    norm = recips[0] + recips[1] + recips[2]
    for k in range(3):
        gid_ref[0, k:k+1, :] = idxs[k] + b * S
        wt_ref[0, k:k+1, :] = recips[k] / norm


@functools.partial(
    pl.kernel, mesh=_sc_mesh, compiler_params=_cp,
    out_type=jax.ShapeDtypeStruct((B, N, C2), jnp.float32),
    scratch_types=[
        pltpu.VMEM((3, PW), jnp.int32),
        pltpu.VMEM((3, PW), jnp.float32),
        pltpu.VMEM((CH, C2), jnp.float32), pltpu.VMEM((CH, C2), jnp.float32),
        pltpu.VMEM((CH, C2), jnp.float32), pltpu.VMEM((CH, C2), jnp.float32),
        pltpu.VMEM((CH, C2), jnp.float32), pltpu.VMEM((CH, C2), jnp.float32),
        pltpu.VMEM((CH, C2), jnp.float32),
        pltpu.SemaphoreType.DMA, pltpu.SemaphoreType.DMA,
        pltpu.SemaphoreType.DMA, pltpu.SemaphoreType.DMA,
        pltpu.SemaphoreType.DMA, pltpu.SemaphoreType.DMA,
    ])
def _sc_gather(table_hbm, gid_hbm, wt_hbm, out_hbm,
               ia, wa, ra0, ra1, ra2, rb0, rb1, rb2, ov,
               sa0, sa1, sa2, sb0, sb1, sb2):
    wid = lax.axis_index("s") * NC + lax.axis_index("c")
    b = wid // 4
    nb0 = (wid % 4) * PW

    pltpu.sync_copy(gid_hbm.at[b, :, pl.ds(nb0, PW)], ia)
    pltpu.sync_copy(wt_hbm.at[b, :, pl.ds(nb0, PW)], wa)

    def fire(g, r0, r1, r2, s0, s1, s2):
        pltpu.async_copy(table_hbm.at[ia.at[0, pl.ds(g * CH, CH)]], r0, s0)
        pltpu.async_copy(table_hbm.at[ia.at[1, pl.ds(g * CH, CH)]], r1, s1)
        pltpu.async_copy(table_hbm.at[ia.at[2, pl.ds(g * CH, CH)]], r2, s2)

    def drain(r0, r1, r2, s0, s1, s2):
        pltpu.make_async_copy(table_hbm.at[pl.ds(0, CH)], r0, s0).wait()
        pltpu.make_async_copy(table_hbm.at[pl.ds(0, CH)], r1, s1).wait()
        pltpu.make_async_copy(table_hbm.at[pl.ds(0, CH)], r2, s2).wait()

    def compute(g, r0, r1, r2):
        @pl.loop(0, CH)
        def _(i):
            p = g * CH + i
            wb0 = plsc.load_gather(
                wa, [jnp.full((L,), 0, jnp.int32), jnp.full((L,), p, jnp.int32)])
            wb1 = plsc.load_gather(
                wa, [jnp.full((L,), 1, jnp.int32), jnp.full((L,), p, jnp.int32)])
            wb2 = plsc.load_gather(
                wa, [jnp.full((L,), 2, jnp.int32), jnp.full((L,), p, jnp.int32)])

            @pl.loop(0, C2, step=L)
            def _(j):
                ov[i, pl.ds(j, L)] = (wb0 * r0[i, pl.ds(j, L)]
                                      + wb1 * r1[i, pl.ds(j, L)]
                                      + wb2 * r2[i, pl.ds(j, L)])

        pltpu.sync_copy(ov, out_hbm.at[b, pl.ds(nb0 + g * CH, CH)])

    fire(0, ra0, ra1, ra2, sa0, sa1, sa2)

    @pl.loop(0, NCH, step=2)
    def _(g):
        fire(g + 1, rb0, rb1, rb2, sb0, sb1, sb2)
        drain(ra0, ra1, ra2, sa0, sa1, sa2)
        compute(g, ra0, ra1, ra2)

        @pl.when(g + 2 < NCH)
        def _():
            fire(g + 2, ra0, ra1, ra2, sa0, sa1, sa2)

        drain(rb0, rb1, rb2, sb0, sb1, sb2)
        compute(g + 1, rb0, rb1, rb2)


def _acc_stats(step, y, acc1, acc2, sto_ref):
    @pl.when(step == 0)
    def _():
        acc1[...] = y
        acc2[...] = y * y

    @pl.when(step != 0)
    def _():
        acc1[...] += y
        acc2[...] += y * y

    @pl.when(step == NSTEPS - 1)
    def _():
        sto_ref[:, 0:1] = jnp.sum(acc1[...], axis=1, keepdims=True)
        sto_ref[:, 1:2] = jnp.sum(acc2[...], axis=1, keepdims=True)


def _l0_body(p1_ref, it_ref, w0a_ref, w0b_ref, b0_ref,
             y_ref, st_ref, acc1, acc2):
    step = pl.program_id(0) * NT + pl.program_id(1)
    y = (jnp.dot(w0a_ref[...], p1_ref[0], preferred_element_type=jnp.float32)
         + lax.dot_general(w0b_ref[...], it_ref[0],
                           (((1,), (1,)), ((), ())),
                           preferred_element_type=jnp.float32)
         + b0_ref[...])
    y_ref[0] = y
    _acc_stats(step, y, acc1, acc2, st_ref)


def _bn_relu(st_ref, g_ref, beta_ref):
    mean = st_ref[:, 0:1] * (1.0 / NPTS)
    var = st_ref[:, 1:2] * (1.0 / NPTS) - mean * mean
    a = g_ref[...] * lax.rsqrt(var + 1e-5)
    c = beta_ref[...] - a * mean
    return a, c


def _layer_body(y_ref, st_ref, g_ref, beta_ref, w_ref, b_ref,
                yo_ref, sto_ref, acc1, acc2):
    step = pl.program_id(0) * NT + pl.program_id(1)
    a, c = _bn_relu(st_ref, g_ref, beta_ref)
    x = jnp.maximum(a * y_ref[0] + c, 0.0)
    y = jnp.dot(w_ref[...], x, preferred_element_type=jnp.float32) + b_ref[...]
    yo_ref[0] = y
    _acc_stats(step, y, acc1, acc2, sto_ref)


def _final_body(y_ref, st_ref, g_ref, beta_ref, o_ref):
    a, c = _bn_relu(st_ref, g_ref, beta_ref)
    o_ref[0] = jnp.maximum(a * y_ref[0] + c, 0.0)


def _col(v):
    return v.reshape(-1, 1)


def kernel(xyz1, xyz2, points1, points2, W0, b0, g0, beta0,
           W1, b1, g1, beta1, W2, b2, g2, beta2):
    xyz2t = jnp.transpose(xyz2, (0, 2, 1))
    p2flat = jnp.transpose(points2, (0, 2, 1)).reshape(B * S, C2)
    w0a, w0b = W0[:, :C1], W0[:, C1:]

    full = lambda shp: pl.BlockSpec(shp, lambda b, n: (0, 0))
    tile = lambda c: pl.BlockSpec((1, c, TN), lambda b, n: (b, 0, n))
    perb = lambda r, c: pl.BlockSpec((1, r, c), lambda b, n: (b, 0, 0))

    gid, wt = pl.pallas_call(
        _dist_body,
        grid=(B, NT),
        in_specs=[tile(3), perb(S, 3)],
        out_specs=[tile(3), tile(3)],
        out_shape=[jax.ShapeDtypeStruct((B, 3, N), jnp.int32),
                   jax.ShapeDtypeStruct((B, 3, N), jnp.float32)],
    )(xyz1, xyz2t)

    interp = _sc_gather(p2flat, gid, wt)

    y0, st0 = pl.pallas_call(
        _l0_body,
        grid=(B, NT),
        in_specs=[tile(C1), pl.BlockSpec((1, TN, C2), lambda b, n: (b, n, 0)),
                  full((512, C1)), full((512, C2)), full((512, 1))],
        out_specs=[tile(512), full((512, 2))],
        out_shape=[jax.ShapeDtypeStruct((B, 512, N), jnp.float32),
                   jax.ShapeDtypeStruct((512, 2), jnp.float32)],
        scratch_shapes=[pltpu.VMEM((512, TN), jnp.float32),
                        pltpu.VMEM((512, TN), jnp.float32)],
    )(points1, interp, w0a, w0b, _col(b0))

    def layer(y, st, g, beta, W, bias, c_in, c_out):
        return pl.pallas_call(
            _layer_body,
            grid=(B, NT),
            in_specs=[tile(c_in), full((c_in, 2)), full((c_in, 1)),
                      full((c_in, 1)), full((c_out, c_in)), full((c_out, 1))],
            out_specs=[tile(c_out), full((c_out, 2))],
            out_shape=[jax.ShapeDtypeStruct((B, c_out, N), jnp.float32),
                       jax.ShapeDtypeStruct((c_out, 2), jnp.float32)],
            scratch_shapes=[pltpu.VMEM((c_out, TN), jnp.float32),
                            pltpu.VMEM((c_out, TN), jnp.float32)],
        )(y, st, _col(g), _col(beta), W, _col(bias))

    y1, st1 = layer(y0, st0, g0, beta0, W1, b1, 512, 512)
    y2, st2 = layer(y1, st1, g1, beta1, W2, b2, 512, 256)

    out = pl.pallas_call(
        _final_body,
        grid=(B, NT),
        in_specs=[tile(256), full((256, 2)), full((256, 1)), full((256, 1))],
        out_specs=tile(256),
        out_shape=jax.ShapeDtypeStruct((B, 256, N), jnp.float32),
    )(y2, st2, _col(g2), _col(beta2))
    return out
